# baseline (speedup 1.0000x reference)
import functools

import jax
import jax.numpy as jnp
from jax import lax
from jax.experimental import pallas as pl
from jax.experimental.pallas import tpu as pltpu
from jax.experimental.pallas import tpu_sc as plsc

_NC = 2
_NS = 16
_NW = _NC * _NS

_CHUNK = 80
_NBUF = 12
_DEPTH = 6


def _gather_rows_sc(x2d, indices, nbatch, n):
    k = indices.shape[0]
    d = x2d.shape[1]
    g = _CHUNK
    lanes = _NW // nbatch
    cpb = k // g
    pc = cpb // lanes
    rem = cpb % lanes
    max_pc = pc + (1 if rem else 0)
    nwin = (max_pc * g) // 16
    mesh = plsc.VectorSubcoreMesh(core_axis_name="c", subcore_axis_name="s")

    @functools.partial(
        pl.kernel,
        out_type=jax.ShapeDtypeStruct((nbatch * k, d), x2d.dtype),
        mesh=mesh,
        scratch_types=(
            [pltpu.VMEM((max_pc * g,), jnp.int32)]
            + [pltpu.VMEM((g, d), x2d.dtype) for _ in range(_NBUF)]
            + [pltpu.SemaphoreType.DMA for _ in range(2 * _NBUF)]
        ),
    )
    def gather_kernel(x_hbm, i_hbm, o_hbm, idx_v, *bufs_and_sems):
        rows = list(bufs_and_sems[:_NBUF])
        sem_g = list(bufs_and_sems[_NBUF:2 * _NBUF])
        sem_s = list(bufs_and_sems[2 * _NBUF:])

        wid = lax.axis_index("s") * _NC + lax.axis_index("c")
        batch = wid // lanes
        lane = wid % lanes
        has_extra = lane < rem
        pos_base = (lane * pc + jnp.minimum(lane, rem)) * g
        row_base = batch * k + pos_base

        pltpu.sync_copy(i_hbm.at[pl.ds(pos_base, pc * g)],
                        idx_v.at[pl.ds(0, pc * g)])

        @pl.when(has_extra)
        def _():
            pltpu.sync_copy(i_hbm.at[pl.ds(pos_base + pc * g, g)],
                            idx_v.at[pl.ds(pc * g, g)])

        off = (batch * n).astype(jnp.int32) + jnp.zeros((16,), jnp.int32)

        @pl.loop(0, nwin)
        def _(w):
            idx_v[pl.ds(w * 16, 16)] = idx_v[pl.ds(w * 16, 16)] + off

        def valid(c):
            return (c < pc) | ((c < max_pc) & has_extra)

        def gather_copy(c, b):
            return pltpu.make_async_copy(
                x_hbm.at[idx_v.at[pl.ds(c * g, g)]], rows[b], sem_g[b])

        def store_copy(c, b):
            return pltpu.make_async_copy(
                rows[b], o_hbm.at[pl.ds(row_base + c * g, g)], sem_s[b])

        for b in range(_DEPTH):
            gather_copy(b, b).start()

        @pl.loop(0, max_pc, step=_NBUF)
        def _(outer):
            for kk in range(_NBUF):
                j = outer + kk
                bk = kk
                b_ahead = (kk + _DEPTH) % _NBUF

                @pl.when((j >= _DEPTH) & valid(j - _DEPTH))
                def _(j=j, b=b_ahead):
                    store_copy(j - _DEPTH, b).wait()

                @pl.when(valid(j + _DEPTH))
                def _(j=j, b=b_ahead):
                    gather_copy(j + _DEPTH, b).start()

                @pl.when(valid(j))
                def _(j=j, b=bk):
                    gather_copy(j, b).wait()
                    store_copy(j, b).start()

        covered = -(-max_pc // _NBUF) * _NBUF
        for c in range(covered - _DEPTH, max_pc):
            @pl.when(valid(c))
            def _(c=c):
                store_copy(c, c % _NBUF).wait()

    return gather_kernel(x2d, indices)


def kernel(x, indices):
    b, n, d = x.shape
    out = _gather_rows_sc(x.reshape(b * n, d), indices, b, n)
    return out.reshape(b, indices.shape[0], d)


# speedup vs baseline: 1.0069x; 1.0069x over previous
import functools

import jax
import jax.numpy as jnp
from jax import lax
from jax.experimental import pallas as pl
from jax.experimental.pallas import tpu as pltpu
from jax.experimental.pallas import tpu_sc as plsc

_NC = 2
_NS = 16
_NW = _NC * _NS

_CHUNK = 80
_NBUF = 12
_DEPTH = 6


def _gather_rows_sc(x2d, indices, nbatch, n):
    k = indices.shape[0]
    d = x2d.shape[1]
    g = _CHUNK
    lanes = _NW // nbatch
    cpb = k // g
    pc = cpb // lanes
    rem = cpb % lanes
    max_pc = pc + (1 if rem else 0)
    nwin = (max_pc * g) // 16
    mesh = plsc.VectorSubcoreMesh(core_axis_name="c", subcore_axis_name="s")

    @functools.partial(
        pl.kernel,
        out_type=jax.ShapeDtypeStruct((nbatch * k, d), x2d.dtype),
        mesh=mesh,
        scratch_types=(
            [pltpu.VMEM((max_pc * g,), jnp.int32)]
            + [pltpu.VMEM((g, d), x2d.dtype) for _ in range(_NBUF)]
            + [pltpu.SemaphoreType.DMA for _ in range(2 * _NBUF)]
        ),
    )
    def gather_kernel(x_hbm, i_hbm, o_hbm, idx_v, *bufs_and_sems):
        rows = list(bufs_and_sems[:_NBUF])
        sem_g = list(bufs_and_sems[_NBUF:2 * _NBUF])
        sem_s = list(bufs_and_sems[2 * _NBUF:])

        wid = lax.axis_index("s") * _NC + lax.axis_index("c")
        batch = wid // lanes
        lane = wid % lanes
        has_extra = lane < rem
        pos_base = (lane * pc + jnp.minimum(lane, rem)) * g
        row_base = batch * k + pos_base

        pltpu.sync_copy(i_hbm.at[pl.ds(pos_base, pc * g)],
                        idx_v.at[pl.ds(0, pc * g)])

        @pl.when(has_extra)
        def _():
            pltpu.sync_copy(i_hbm.at[pl.ds(pos_base + pc * g, g)],
                            idx_v.at[pl.ds(pc * g, g)])

        x_batch = x_hbm.at[pl.ds(batch * n, n)]

        def valid(c):
            return (c < pc) | ((c < max_pc) & has_extra)

        def gather_copy(c, b):
            return pltpu.make_async_copy(
                x_batch.at[idx_v.at[pl.ds(c * g, g)]], rows[b], sem_g[b])

        def store_copy(c, b):
            return pltpu.make_async_copy(
                rows[b], o_hbm.at[pl.ds(row_base + c * g, g)], sem_s[b])

        for b in range(_DEPTH):
            gather_copy(b, b).start()

        @pl.loop(0, max_pc, step=_NBUF)
        def _(outer):
            for kk in range(_NBUF):
                j = outer + kk
                bk = kk
                b_ahead = (kk + _DEPTH) % _NBUF

                @pl.when((j >= _DEPTH) & valid(j - _DEPTH))
                def _(j=j, b=b_ahead):
                    store_copy(j - _DEPTH, b).wait()

                @pl.when(valid(j + _DEPTH))
                def _(j=j, b=b_ahead):
                    gather_copy(j + _DEPTH, b).start()

                @pl.when(valid(j))
                def _(j=j, b=bk):
                    gather_copy(j, b).wait()
                    store_copy(j, b).start()

        covered = -(-max_pc // _NBUF) * _NBUF
        for c in range(covered - _DEPTH, max_pc):
            @pl.when(valid(c))
            def _(c=c):
                store_copy(c, c % _NBUF).wait()

    return gather_kernel(x2d, indices)


def kernel(x, indices):
    b, n, d = x.shape
    out = _gather_rows_sc(x.reshape(b * n, d), indices, b, n)
    return out.reshape(b, indices.shape[0], d)


# speedup vs baseline: 1.0134x; 1.0064x over previous
import functools

import jax
import jax.numpy as jnp
from jax import lax
from jax.experimental import pallas as pl
from jax.experimental.pallas import tpu as pltpu
from jax.experimental.pallas import tpu_sc as plsc

_NC = 2
_NS = 16
_NW = _NC * _NS

_CHUNK = 80
_NBUF = 12
_DEPTH = 6


def _gather_rows_sc(x2d, indices, nbatch, n):
    k = indices.shape[0]
    d = x2d.shape[1]
    g = _CHUNK
    lanes = _NW // nbatch
    cpb = k // g
    pc = cpb // lanes
    rem = cpb % lanes
    max_pc = pc + (1 if rem else 0)
    mesh = plsc.VectorSubcoreMesh(core_axis_name="c", subcore_axis_name="s")

    @functools.partial(
        pl.kernel,
        out_type=jax.ShapeDtypeStruct((nbatch * k, d), x2d.dtype),
        mesh=mesh,
        scratch_types=(
            [pltpu.VMEM((max_pc * g,), jnp.int32)]
            + [pltpu.VMEM((g, d), x2d.dtype) for _ in range(_NBUF)]
            + [pltpu.SemaphoreType.DMA for _ in range(2 * _NBUF)]
        ),
    )
    def gather_kernel(x_hbm, i_hbm, o_hbm, idx_v, *bufs_and_sems):
        rows = list(bufs_and_sems[:_NBUF])
        sem_g = list(bufs_and_sems[_NBUF:2 * _NBUF])
        sem_s = list(bufs_and_sems[2 * _NBUF:])

        wid = lax.axis_index("s") * _NC + lax.axis_index("c")
        batch = wid // lanes
        lane = wid % lanes
        has_extra = lane < rem
        pos_base = (lane * pc + jnp.minimum(lane, rem)) * g
        row_base = batch * k + pos_base

        pltpu.sync_copy(i_hbm.at[pl.ds(pos_base, pc * g)],
                        idx_v.at[pl.ds(0, pc * g)])

        @pl.when(has_extra)
        def _():
            pltpu.sync_copy(i_hbm.at[pl.ds(pos_base + pc * g, g)],
                            idx_v.at[pl.ds(pc * g, g)])

        x_batch = x_hbm.at[pl.ds(batch * n, n)]

        def valid(c):
            return (c < pc) | ((c < max_pc) & has_extra)

        def gather_copy(c, b):
            return pltpu.make_async_copy(
                x_batch.at[idx_v.at[pl.ds(c * g, g)]], rows[b], sem_g[b])

        def store_copy(c, b):
            return pltpu.make_async_copy(
                rows[b], o_hbm.at[pl.ds(row_base + c * g, g)], sem_s[b])

        for b in range(_DEPTH):
            gather_copy(b, b).start()

        @pl.loop(0, max_pc, step=_NBUF)
        def _(outer):
            for kk in range(_NBUF):
                j = outer + kk
                bk = kk
                b_ahead = (kk + _DEPTH) % _NBUF

                @pl.when((j >= _DEPTH) & valid(j - _DEPTH))
                def _(j=j, b=b_ahead):
                    store_copy(j - _DEPTH, b).wait()

                @pl.when(valid(j + _DEPTH))
                def _(j=j, b=b_ahead):
                    gather_copy(j + _DEPTH, b).start()

                @pl.when(valid(j))
                def _(j=j, b=bk):
                    gather_copy(j, b).wait()
                    store_copy(j, b).start()

        covered = -(-max_pc // _NBUF) * _NBUF
        for c in range(covered - _DEPTH, max_pc):
            @pl.when(valid(c))
            def _(c=c):
                store_copy(c, c % _NBUF).wait()

    return gather_kernel(x2d, indices)


def kernel(x, indices):
    b, n, d = x.shape
    out = _gather_rows_sc(x.reshape(b * n, d), indices, b, n)
    return out.reshape(b, indices.shape[0], d)
